# baseline (speedup 1.0000x reference)
import functools

import jax
import jax.numpy as jnp
from jax import lax
from jax.experimental import pallas as pl
from jax.experimental.pallas import tpu as pltpu
from jax.experimental.pallas import tpu_sc as plsc

L = 16
NC, NS = 2, 16
NW = NC * NS
B = 16384
F = 64
BPW = B // NW
GROUPS = BPW // L

_mesh = plsc.VectorSubcoreMesh(core_axis_name="c", subcore_axis_name="s")


@functools.partial(
    pl.kernel,
    out_type=jax.ShapeDtypeStruct((B,), jnp.float32),
    mesh=_mesh,
    compiler_params=pltpu.CompilerParams(
        needs_layout_passes=False, use_tc_tiling_on_sc=False
    ),
    scratch_types=[
        pltpu.VMEM((BPW,), jnp.int32),
        pltpu.VMEM((BPW,), jnp.int32),
        pltpu.VMEM((BPW, F), jnp.float32),
        pltpu.VMEM((BPW, F), jnp.float32),
        pltpu.VMEM((BPW,), jnp.float32),
        pltpu.VMEM((BPW,), jnp.float32),
        pltpu.VMEM((BPW,), jnp.float32),
        pltpu.VMEM((L * (L + 1),), jnp.float32),
        pltpu.SemaphoreType.DMA,
    ],
)
def _mf_kernel(users_hbm, items_hbm, uf_hbm, if_hbm, ub_hbm, ib_hbm, out_hbm,
               uidx, iidx, urows, irows, ubv, ibv, outv, tbuf, sem):
    wid = lax.axis_index("s") * NC + lax.axis_index("c")
    base = wid * BPW

    pltpu.sync_copy(users_hbm.at[pl.ds(base, BPW)], uidx)
    pltpu.sync_copy(items_hbm.at[pl.ds(base, BPW)], iidx)

    c1 = pltpu.async_copy(uf_hbm.at[uidx], urows, sem)
    c2 = pltpu.async_copy(if_hbm.at[iidx], irows, sem)
    c3 = pltpu.async_copy(ub_hbm.at[uidx], ubv, sem)
    c4 = pltpu.async_copy(ib_hbm.at[iidx], ibv, sem)
    c1.wait()
    c2.wait()
    c3.wait()
    c4.wait()

    rowi = lax.iota(jnp.int32, L)

    def group_body(g, carry):
        gb = g * L
        for r in range(L):
            b = gb + r
            acc = urows[b, pl.ds(0, L)] * irows[b, pl.ds(0, L)]
            for j in range(1, F // L):
                acc = acc + urows[b, pl.ds(j * L, L)] * irows[b, pl.ds(j * L, L)]
            tbuf[pl.ds(r * (L + 1), L)] = acc
        out16 = ubv[pl.ds(gb, L)] + ibv[pl.ds(gb, L)]
        flat = rowi * (L + 1)
        for j in range(L):
            out16 = out16 + plsc.load_gather(tbuf, [flat + j])
        outv[pl.ds(gb, L)] = out16
        return carry

    lax.fori_loop(0, GROUPS, group_body, 0)
    pltpu.sync_copy(outv, out_hbm.at[pl.ds(base, BPW)])


@jax.jit
def kernel(users, items, user_factors, item_factors, user_bias, item_bias):
    ub = user_bias.reshape(-1)
    ib = item_bias.reshape(-1)
    return _mf_kernel(users, items, user_factors, item_factors, ub, ib)
